# baseline (speedup 1.0000x reference)
import functools

import jax
import jax.numpy as jnp
from jax import lax
from jax.experimental import pallas as pl
from jax.experimental.pallas import tpu as pltpu
from jax.experimental.pallas import tpu_sc as plsc

DIM = 64
HB = 128
GH = 2
K0_W = 512


@functools.lru_cache(maxsize=None)
def _make_k0(V: int):
    nblk = (V + K0_W - 1) // K0_W
    Vp = nblk * K0_W

    def body(wt_ref, o_ref):
        x = wt_ref[...]
        o_ref[...] = jnp.concatenate(
            [x.T, jnp.zeros((K0_W, 128 - DIM), jnp.float32)], axis=1
        )

    return pl.pallas_call(
        body,
        grid=(nblk,),
        in_specs=[pl.BlockSpec((DIM, K0_W), lambda i: (0, i))],
        out_specs=pl.BlockSpec((K0_W, 128), lambda i: (i, 0)),
        out_shape=jax.ShapeDtypeStruct((Vp, 128), jnp.float32),
        compiler_params=pltpu.CompilerParams(
            dimension_semantics=("arbitrary",)
        ),
    )


@functools.lru_cache(maxsize=None)
def _make_k1(B: int, H: int, Vp: int):
    info = plsc.get_sparse_core_info()
    nc = info.num_cores
    nw = nc * info.num_subcores
    blk_per_w = B // (HB * nw)
    idx_per_blk = HB * H
    steps_per_blk = H // GH
    nsteps = blk_per_w * steps_per_blk
    rows = GH * HB
    assert B % (HB * nw) == 0 and H % GH == 0
    mesh = plsc.VectorSubcoreMesh(core_axis_name="c", subcore_axis_name="s")

    @functools.partial(
        pl.kernel,
        mesh=mesh,
        out_type=jax.ShapeDtypeStruct((H, B, 128), jnp.float32),
        scratch_types=[
            pltpu.VMEM((idx_per_blk,), jnp.int32),
            pltpu.VMEM((blk_per_w * idx_per_blk,), jnp.int32),
            pltpu.VMEM((3 * rows, 128), jnp.float32),
            pltpu.SemaphoreType.DMA,
            pltpu.SemaphoreType.DMA,
        ],
        compiler_params=pltpu.CompilerParams(
            use_tc_tiling_on_sc=False, needs_layout_passes=False
        ),
    )
    def k1(idx_hbm, table_hbm, out_hbm, idx_v, idxT, gath, gsem, osem):
        wid = lax.axis_index("s") * nc + lax.axis_index("c")
        base = wid * blk_per_w * idx_per_blk
        iota = lax.iota(jnp.int32, 16)

        for blk in range(blk_per_w):
            pltpu.sync_copy(
                idx_hbm.at[pl.ds(base + blk * idx_per_blk, idx_per_blk)], idx_v
            )

            def reorder(h, _, blk=blk):
                for j in range(HB // 16):
                    v = plsc.load_gather(idx_v, [iota * H + (j * 16 * H + h)])
                    idxT[pl.ds(blk * idx_per_blk + h * HB + j * 16, 16)] = v
                return 0

            lax.fori_loop(0, H, reorder, 0)

        def start_gather(g, p):
            pltpu.async_copy(
                table_hbm.at[idxT.at[pl.ds(g * rows, rows)]],
                gath.at[pl.ds(p * rows, rows)],
                gsem,
            )

        def start_out(g, p, hh):
            blk = g // steps_per_blk
            h0 = (g % steps_per_blk) * GH
            pltpu.async_copy(
                gath.at[pl.ds(p * rows + hh * HB, HB)],
                out_hbm.at[h0 + hh, pl.ds((wid * blk_per_w + blk) * HB, HB)],
                osem,
            )

        def wait_bytes(sem, n):
            pltpu.make_async_copy(
                gath.at[pl.ds(0, n)], out_hbm.at[0, pl.ds(0, n)], sem
            ).wait()

        start_gather(0, 0)

        def body(g, _):
            p = lax.rem(g, 3)
            pl.when(g >= 2)(lambda: wait_bytes(osem, rows))
            pl.when(g + 1 < nsteps)(
                lambda: start_gather(g + 1, lax.rem(g + 1, 3)))
            wait_bytes(gsem, rows)
            for hh in range(GH):
                start_out(g, p, hh)
            return 0

        lax.fori_loop(0, nsteps, body, 0)
        wait_bytes(osem, rows)
        wait_bytes(osem, rows)

    return k1


@functools.lru_cache(maxsize=None)
def _make_k2(B: int, H: int):
    def body(x_ref, o_ref):
        x = x_ref[0, :, : DIM]
        o_ref[0, :, 0] = x.T.reshape(DIM // 8, 8, HB)

    return pl.pallas_call(
        body,
        grid=(H, B // HB),
        in_specs=[pl.BlockSpec((1, HB, 128), lambda i, j: (i, j, 0))],
        out_specs=pl.BlockSpec(
            (1, DIM // 8, 1, 8, HB), lambda i, j: (i, 0, j, 0, 0)
        ),
        out_shape=jax.ShapeDtypeStruct((H, DIM // 8, B // HB, 8, HB), jnp.float32),
        compiler_params=pltpu.CompilerParams(
            dimension_semantics=("arbitrary", "arbitrary")
        ),
    )


def kernel(input_, weight):
    b, h = input_.shape
    v = weight.shape[0]
    idx = input_.reshape(b * h).astype(jnp.int32)
    t128 = _make_k0(v)(weight.T)
    inter = _make_k1(b, h, t128.shape[0])(idx, t128)
    out5 = _make_k2(b, h)(inter)
    return out5.transpose(2, 4, 0, 1, 3).reshape(b, h, DIM)


# speedup vs baseline: 4.2032x; 4.2032x over previous
import functools

import jax
import jax.numpy as jnp
from jax import lax
from jax.experimental import pallas as pl
from jax.experimental.pallas import tpu as pltpu
from jax.experimental.pallas import tpu_sc as plsc

DIM = 64
HB = 128
GH = 2
K0_W = 8192


def _mxu_t(x):
    n = x.shape[0]
    eye = jnp.eye(n, dtype=jnp.float32)
    return jax.lax.dot_general(
        x, eye, (((0,), (0,)), ((), ())),
        precision=jax.lax.Precision.HIGHEST,
        preferred_element_type=jnp.float32,
    )


@functools.lru_cache(maxsize=None)
def _make_k0(V: int):
    nblk = (V + K0_W - 1) // K0_W
    Vp = nblk * K0_W

    def body(wt_ref, o_ref):
        x = wt_ref[...]
        o_ref[...] = jnp.concatenate(
            [_mxu_t(x), jnp.zeros((K0_W, 128 - DIM), jnp.float32)], axis=1
        )

    return pl.pallas_call(
        body,
        grid=(nblk,),
        in_specs=[pl.BlockSpec((DIM, K0_W), lambda i: (0, i))],
        out_specs=pl.BlockSpec((K0_W, 128), lambda i: (i, 0)),
        out_shape=jax.ShapeDtypeStruct((Vp, 128), jnp.float32),
        compiler_params=pltpu.CompilerParams(
            dimension_semantics=("arbitrary",)
        ),
    )


@functools.lru_cache(maxsize=None)
def _make_k1(B: int, H: int, Vp: int):
    info = plsc.get_sparse_core_info()
    nc = info.num_cores
    nw = nc * info.num_subcores
    blk_per_w = B // (HB * nw)
    idx_per_blk = HB * H
    steps_per_blk = H // GH
    nsteps = blk_per_w * steps_per_blk
    rows = GH * HB
    assert B % (HB * nw) == 0 and H % GH == 0
    mesh = plsc.VectorSubcoreMesh(core_axis_name="c", subcore_axis_name="s")

    @functools.partial(
        pl.kernel,
        mesh=mesh,
        out_type=jax.ShapeDtypeStruct((H, B, 128), jnp.float32),
        scratch_types=[
            pltpu.VMEM((idx_per_blk,), jnp.int32),
            pltpu.VMEM((blk_per_w * idx_per_blk,), jnp.int32),
            pltpu.VMEM((3 * rows, 128), jnp.float32),
            pltpu.SemaphoreType.DMA,
            pltpu.SemaphoreType.DMA,
        ],
        compiler_params=pltpu.CompilerParams(
            use_tc_tiling_on_sc=False, needs_layout_passes=False
        ),
    )
    def k1(idx_hbm, table_hbm, out_hbm, idx_v, idxT, gath, gsem, osem):
        wid = lax.axis_index("s") * nc + lax.axis_index("c")
        base = wid * blk_per_w * idx_per_blk
        iota = lax.iota(jnp.int32, 16)

        for blk in range(blk_per_w):
            pltpu.sync_copy(
                idx_hbm.at[pl.ds(base + blk * idx_per_blk, idx_per_blk)], idx_v
            )

            def reorder(h, _, blk=blk):
                for j in range(HB // 16):
                    v = plsc.load_gather(idx_v, [iota * H + (j * 16 * H + h)])
                    idxT[pl.ds(blk * idx_per_blk + h * HB + j * 16, 16)] = v
                return 0

            lax.fori_loop(0, H, reorder, 0)

        def start_gather(g, p):
            pltpu.async_copy(
                table_hbm.at[idxT.at[pl.ds(g * rows, rows)]],
                gath.at[pl.ds(p * rows, rows)],
                gsem,
            )

        def start_out(g, p, hh):
            blk = g // steps_per_blk
            h0 = (g % steps_per_blk) * GH
            pltpu.async_copy(
                gath.at[pl.ds(p * rows + hh * HB, HB)],
                out_hbm.at[h0 + hh, pl.ds((wid * blk_per_w + blk) * HB, HB)],
                osem,
            )

        def wait_bytes(sem, n):
            pltpu.make_async_copy(
                gath.at[pl.ds(0, n)], out_hbm.at[0, pl.ds(0, n)], sem
            ).wait()

        start_gather(0, 0)

        def body(g, _):
            p = lax.rem(g, 3)
            pl.when(g >= 2)(lambda: wait_bytes(osem, rows))
            pl.when(g + 1 < nsteps)(
                lambda: start_gather(g + 1, lax.rem(g + 1, 3)))
            wait_bytes(gsem, rows)
            for hh in range(GH):
                start_out(g, p, hh)
            return 0

        lax.fori_loop(0, nsteps, body, 0)
        wait_bytes(osem, rows)
        wait_bytes(osem, rows)

    return k1


@functools.lru_cache(maxsize=None)
def _make_k2(B: int, H: int):
    NB = 32

    def body(x_ref, o_ref):
        for k in range(NB):
            x = x_ref[0, pl.ds(k * HB, HB), : DIM]
            o_ref[0, :, k] = _mxu_t(x).reshape(DIM // 8, 8, HB)

    return pl.pallas_call(
        body,
        grid=(H, B // (HB * NB)),
        in_specs=[pl.BlockSpec((1, HB * NB, 128), lambda i, j: (i, j, 0))],
        out_specs=pl.BlockSpec(
            (1, DIM // 8, NB, 8, HB), lambda i, j: (i, 0, j, 0, 0)
        ),
        out_shape=jax.ShapeDtypeStruct((H, DIM // 8, B // HB, 8, HB), jnp.float32),
        compiler_params=pltpu.CompilerParams(
            dimension_semantics=("arbitrary", "arbitrary")
        ),
    )


def kernel(input_, weight):
    b, h = input_.shape
    v = weight.shape[0]
    idx = input_.reshape(b * h).astype(jnp.int32)
    t128 = _make_k0(v)(weight.T)
    inter = _make_k1(b, h, t128.shape[0])(idx, t128)
    out5 = _make_k2(b, h)(inter)
    return out5.transpose(2, 4, 0, 1, 3).reshape(b, h, DIM)


# speedup vs baseline: 4.6049x; 1.0956x over previous
import functools

import jax
import jax.numpy as jnp
from jax import lax
from jax.experimental import pallas as pl
from jax.experimental.pallas import tpu as pltpu
from jax.experimental.pallas import tpu_sc as plsc

DIM = 64
HB = 128
GH = 2
K0_W = 8192
HC = 10


def _mxu_t(x):
    n = x.shape[0]
    eye = jnp.eye(n, dtype=jnp.float32)
    return jax.lax.dot_general(
        x, eye, (((0,), (0,)), ((), ())),
        precision=jax.lax.Precision.HIGHEST,
        preferred_element_type=jnp.float32,
    )


@functools.lru_cache(maxsize=None)
def _make_k0(V: int):
    nblk = (V + K0_W - 1) // K0_W
    Vp = nblk * K0_W

    def body(wt_ref, o_ref):
        x = wt_ref[...]
        o_ref[...] = jnp.concatenate(
            [_mxu_t(x), jnp.zeros((K0_W, 128 - DIM), jnp.float32)], axis=1
        )

    return pl.pallas_call(
        body,
        grid=(nblk,),
        in_specs=[pl.BlockSpec((DIM, K0_W), lambda i: (0, i))],
        out_specs=pl.BlockSpec((K0_W, 128), lambda i: (i, 0)),
        out_shape=jax.ShapeDtypeStruct((Vp, 128), jnp.float32),
        compiler_params=pltpu.CompilerParams(
            dimension_semantics=("arbitrary",)
        ),
    )


@functools.lru_cache(maxsize=None)
def _make_k1(B: int, H: int, h0: int):
    info = plsc.get_sparse_core_info()
    nc = info.num_cores
    nw = nc * info.num_subcores
    blk_per_w = B // (HB * nw)
    idx_per_blk = HB * H
    steps_per_blk = HC // GH
    nsteps = blk_per_w * steps_per_blk
    rows = GH * HB
    assert B % (HB * nw) == 0 and HC % GH == 0
    mesh = plsc.VectorSubcoreMesh(core_axis_name="c", subcore_axis_name="s")

    @functools.partial(
        pl.kernel,
        mesh=mesh,
        out_type=jax.ShapeDtypeStruct((HC, B, 128), jnp.float32),
        scratch_types=[
            pltpu.VMEM((idx_per_blk,), jnp.int32),
            pltpu.VMEM((blk_per_w * HC * HB,), jnp.int32),
            pltpu.VMEM((3 * rows, 128), jnp.float32),
            pltpu.SemaphoreType.DMA,
            pltpu.SemaphoreType.DMA,
        ],
        compiler_params=pltpu.CompilerParams(
            use_tc_tiling_on_sc=False, needs_layout_passes=False
        ),
    )
    def k1(idx_hbm, table_hbm, out_hbm, idx_v, idxT, gath, gsem, osem):
        wid = lax.axis_index("s") * nc + lax.axis_index("c")
        base = wid * blk_per_w * idx_per_blk
        iota = lax.iota(jnp.int32, 16)

        for blk in range(blk_per_w):
            pltpu.sync_copy(
                idx_hbm.at[pl.ds(base + blk * idx_per_blk, idx_per_blk)], idx_v
            )

            def reorder(hl, _, blk=blk):
                for j in range(HB // 16):
                    v = plsc.load_gather(
                        idx_v, [iota * H + (j * 16 * H + h0 + hl)])
                    idxT[pl.ds(blk * HC * HB + hl * HB + j * 16, 16)] = v
                return 0

            lax.fori_loop(0, HC, reorder, 0)

        def start_gather(g, p):
            pltpu.async_copy(
                table_hbm.at[idxT.at[pl.ds(g * rows, rows)]],
                gath.at[pl.ds(p * rows, rows)],
                gsem,
            )

        def start_out(g, p, hh):
            blk = g // steps_per_blk
            hl = (g % steps_per_blk) * GH + hh
            pltpu.async_copy(
                gath.at[pl.ds(p * rows + hh * HB, HB)],
                out_hbm.at[hl, pl.ds((wid * blk_per_w + blk) * HB, HB)],
                osem,
            )

        def wait_bytes(sem, n):
            pltpu.make_async_copy(
                gath.at[pl.ds(0, n)], out_hbm.at[0, pl.ds(0, n)], sem
            ).wait()

        start_gather(0, 0)

        def body(g, _):
            p = lax.rem(g, 3)
            pl.when(g >= 2)(lambda: wait_bytes(osem, rows))
            pl.when(g + 1 < nsteps)(
                lambda: start_gather(g + 1, lax.rem(g + 1, 3)))
            wait_bytes(gsem, rows)
            for hh in range(GH):
                start_out(g, p, hh)
            return 0

        lax.fori_loop(0, nsteps, body, 0)
        wait_bytes(osem, rows)
        wait_bytes(osem, rows)

    return k1


@functools.lru_cache(maxsize=None)
def _make_k2(B: int, H: int, h0: int, aliased: bool):
    NB = 32

    def body(*refs):
        x_ref = refs[0]
        o_ref = refs[-1]
        for k in range(NB):
            x = x_ref[0, pl.ds(k * HB, HB), : DIM]
            o_ref[0, :, k] = _mxu_t(x).reshape(DIM // 8, 8, HB)

    out_shape = jax.ShapeDtypeStruct((H, DIM // 8, B // HB, 8, HB), jnp.float32)
    in_specs = [pl.BlockSpec((1, HB * NB, 128), lambda i, j: (i, j, 0))]
    num_inputs = 1
    kwargs = {}
    if aliased:
        in_specs.append(pl.BlockSpec(memory_space=pl.ANY))
        num_inputs = 2
        kwargs["input_output_aliases"] = {1: 0}
    return pl.pallas_call(
        body,
        grid=(HC, B // (HB * NB)),
        in_specs=in_specs,
        out_specs=pl.BlockSpec(
            (1, DIM // 8, NB, 8, HB), lambda i, j: (h0 + i, 0, j, 0, 0)
        ),
        out_shape=out_shape,
        compiler_params=pltpu.CompilerParams(
            dimension_semantics=("arbitrary", "arbitrary")
        ),
        **kwargs,
    )


def kernel(input_, weight):
    b, h = input_.shape
    v = weight.shape[0]
    idx = input_.reshape(b * h).astype(jnp.int32)
    t128 = _make_k0(v)(weight.T)
    slabs = [
        _make_k1(b, h, h0)(idx, t128) for h0 in range(0, h, HC)
    ]
    out5 = _make_k2(b, h, 0, False)(slabs[0])
    for i, h0 in enumerate(range(HC, h, HC)):
        out5 = _make_k2(b, h, h0, True)(slabs[i + 1], out5)
    return out5.transpose(2, 4, 0, 1, 3).reshape(b, h, DIM)


# speedup vs baseline: 4.6053x; 1.0001x over previous
import functools

import jax
import jax.numpy as jnp
from jax import lax
from jax.experimental import pallas as pl
from jax.experimental.pallas import tpu as pltpu
from jax.experimental.pallas import tpu_sc as plsc

DIM = 64
HB = 128
GH = 2
K0_W = 8192
HC = 10


def _mxu_t(x):
    n = x.shape[0]
    eye = jnp.eye(n, dtype=jnp.float32)
    return jax.lax.dot_general(
        x, eye, (((0,), (0,)), ((), ())),
        precision=jax.lax.Precision.HIGHEST,
        preferred_element_type=jnp.float32,
    )


@functools.lru_cache(maxsize=None)
def _make_k0(V: int):
    nblk = (V + K0_W - 1) // K0_W
    Vp = nblk * K0_W

    def body(wt_ref, o_ref):
        x = wt_ref[...]
        o_ref[...] = jnp.concatenate(
            [_mxu_t(x), jnp.zeros((K0_W, 128 - DIM), jnp.float32)], axis=1
        )

    return pl.pallas_call(
        body,
        grid=(nblk,),
        in_specs=[pl.BlockSpec((DIM, K0_W), lambda i: (0, i))],
        out_specs=pl.BlockSpec((K0_W, 128), lambda i: (i, 0)),
        out_shape=jax.ShapeDtypeStruct((Vp, 128), jnp.float32),
        compiler_params=pltpu.CompilerParams(
            dimension_semantics=("arbitrary",)
        ),
    )


@functools.lru_cache(maxsize=None)
def _make_k1(B: int, H: int, h0: int):
    info = plsc.get_sparse_core_info()
    nc = info.num_cores
    nw = nc * info.num_subcores
    blk_per_w = B // (HB * nw)
    idx_per_blk = HB * H
    steps_per_blk = HC // GH
    nsteps = blk_per_w * steps_per_blk
    rows = GH * HB
    assert B % (HB * nw) == 0 and HC % GH == 0
    mesh = plsc.VectorSubcoreMesh(core_axis_name="c", subcore_axis_name="s")

    @functools.partial(
        pl.kernel,
        mesh=mesh,
        out_type=jax.ShapeDtypeStruct((HC, B, 128), jnp.float32),
        scratch_types=[
            pltpu.VMEM((idx_per_blk,), jnp.int32),
            pltpu.VMEM((blk_per_w * HC * HB,), jnp.int32),
            pltpu.VMEM((3 * rows, 128), jnp.float32),
            pltpu.SemaphoreType.DMA,
            pltpu.SemaphoreType.DMA,
        ],
        compiler_params=pltpu.CompilerParams(
            use_tc_tiling_on_sc=False, needs_layout_passes=False
        ),
    )
    def k1(idx_hbm, table_hbm, out_hbm, idx_v, idxT, gath, gsem, osem):
        wid = lax.axis_index("s") * nc + lax.axis_index("c")
        base = wid * blk_per_w * idx_per_blk
        iota = lax.iota(jnp.int32, 16)

        for blk in range(blk_per_w):
            pltpu.sync_copy(
                idx_hbm.at[pl.ds(base + blk * idx_per_blk, idx_per_blk)], idx_v
            )

            def reorder(hl, _, blk=blk):
                for j in range(HB // 16):
                    v = plsc.load_gather(
                        idx_v, [iota * H + (j * 16 * H + h0 + hl)])
                    idxT[pl.ds(blk * HC * HB + hl * HB + j * 16, 16)] = v
                return 0

            lax.fori_loop(0, HC, reorder, 0)

        def start_gather(g, p):
            pltpu.async_copy(
                table_hbm.at[idxT.at[pl.ds(g * rows, rows)]],
                gath.at[pl.ds(p * rows, rows)],
                gsem,
            )

        def start_out(g, p, hh):
            blk = g // steps_per_blk
            hl = (g % steps_per_blk) * GH + hh
            pltpu.async_copy(
                gath.at[pl.ds(p * rows + hh * HB, HB)],
                out_hbm.at[hl, pl.ds((wid * blk_per_w + blk) * HB, HB)],
                osem,
            )

        def wait_bytes(sem, n):
            pltpu.make_async_copy(
                gath.at[pl.ds(0, n)], out_hbm.at[0, pl.ds(0, n)], sem
            ).wait()

        start_gather(0, 0)

        def body(g, _):
            p = lax.rem(g, 3)
            pl.when(g >= 2)(lambda: wait_bytes(osem, rows))
            pl.when(g + 1 < nsteps)(
                lambda: start_gather(g + 1, lax.rem(g + 1, 3)))
            wait_bytes(gsem, rows)
            for hh in range(GH):
                start_out(g, p, hh)
            return 0

        lax.fori_loop(0, nsteps, body, 0)
        wait_bytes(osem, rows)
        wait_bytes(osem, rows)

    return k1


@functools.lru_cache(maxsize=None)
def _make_k2(B: int, H: int, h0: int, aliased: bool):
    NB = 32

    def body(*refs):
        x_ref = refs[0]
        o_ref = refs[-1]
        for k in range(NB):
            x = x_ref[0, pl.ds(k * HB, HB), : DIM]
            o_ref[0, :, k] = _mxu_t(x).reshape(DIM // 8, 8, HB)

    out_shape = jax.ShapeDtypeStruct((H, DIM // 8, B // HB, 8, HB), jnp.float32)
    in_specs = [
        pl.BlockSpec((1, HB * NB, 128), lambda i, j: (i, j, 0)),
    ]
    kwargs = {}
    if aliased:
        in_specs.append(pl.BlockSpec(memory_space=pl.ANY))
        kwargs["input_output_aliases"] = {1: 0}
    return pl.pallas_call(
        body,
        grid=(HC, B // (HB * NB)),
        in_specs=in_specs,
        out_specs=pl.BlockSpec(
            (1, DIM // 8, NB, 8, HB), lambda i, j: (h0 + i, 0, j, 0, 0)
        ),
        out_shape=out_shape,
        compiler_params=pltpu.CompilerParams(
            dimension_semantics=("arbitrary", "arbitrary")
        ),
        **kwargs,
    )


def kernel(input_, weight):
    b, h = input_.shape
    v = weight.shape[0]
    idx = input_.reshape(b * h).astype(jnp.int32)
    t128 = _make_k0(v)(weight.T)
    slabs = [_make_k1(b, h, h0)(idx, t128) for h0 in range(0, h, HC)]
    out5 = _make_k2(b, h, 0, False)(slabs[0])
    for i, h0 in enumerate(range(HC, h, HC)):
        out5 = _make_k2(b, h, h0, True)(slabs[i + 1], out5)
    return out5.transpose(2, 4, 0, 1, 3).reshape(b, h, DIM)


# speedup vs baseline: 4.7859x; 1.0392x over previous
import functools

import jax
import jax.numpy as jnp
from jax import lax
from jax.experimental import pallas as pl
from jax.experimental.pallas import tpu as pltpu
from jax.experimental.pallas import tpu_sc as plsc

DIM = 64
HB = 128
GH = 2
K0_W = 8192
HC = 10


def _mxu_t(x):
    n = x.shape[0]
    eye = jnp.eye(n, dtype=jnp.float32)
    return jax.lax.dot_general(
        x, eye, (((0,), (0,)), ((), ())),
        precision=jax.lax.Precision.HIGHEST,
        preferred_element_type=jnp.float32,
    )


@functools.lru_cache(maxsize=None)
def _make_k0(V: int):
    nblk = (V + K0_W - 1) // K0_W
    Vp = nblk * K0_W

    def body(wt_ref, o_ref):
        x = wt_ref[...]
        o_ref[...] = jnp.concatenate(
            [_mxu_t(x), jnp.zeros((K0_W, 128 - DIM), jnp.float32)], axis=1
        )

    return pl.pallas_call(
        body,
        grid=(nblk,),
        in_specs=[pl.BlockSpec((DIM, K0_W), lambda i: (0, i))],
        out_specs=pl.BlockSpec((K0_W, 128), lambda i: (i, 0)),
        out_shape=jax.ShapeDtypeStruct((Vp, 128), jnp.float32),
        compiler_params=pltpu.CompilerParams(
            dimension_semantics=("arbitrary",)
        ),
    )


@functools.lru_cache(maxsize=None)
def _make_k1(B: int, H: int, h0: int):
    info = plsc.get_sparse_core_info()
    nc = info.num_cores
    nw = nc * info.num_subcores
    blk_per_w = B // (HB * nw)
    idx_per_blk = HB * H
    steps_per_blk = HC // GH
    nsteps = blk_per_w * steps_per_blk
    rows = GH * HB
    assert B % (HB * nw) == 0 and HC % GH == 0
    mesh = plsc.VectorSubcoreMesh(core_axis_name="c", subcore_axis_name="s")

    @functools.partial(
        pl.kernel,
        mesh=mesh,
        out_type=jax.ShapeDtypeStruct((HC, B, 128), jnp.float32),
        scratch_types=[
            pltpu.VMEM((idx_per_blk,), jnp.int32),
            pltpu.VMEM((blk_per_w * HC * HB,), jnp.int32),
            pltpu.VMEM((3 * rows, 128), jnp.float32),
            pltpu.SemaphoreType.DMA,
            pltpu.SemaphoreType.DMA,
        ],
        compiler_params=pltpu.CompilerParams(
            use_tc_tiling_on_sc=False, needs_layout_passes=False
        ),
    )
    def k1(idx_hbm, table_hbm, out_hbm, idx_v, idxT, gath, gsem, osem):
        wid = lax.axis_index("s") * nc + lax.axis_index("c")
        base = wid * blk_per_w * idx_per_blk
        iota = lax.iota(jnp.int32, 16)

        for blk in range(blk_per_w):
            pltpu.sync_copy(
                idx_hbm.at[pl.ds(base + blk * idx_per_blk, idx_per_blk)], idx_v
            )

            def reorder(hl, _, blk=blk):
                for j in range(HB // 16):
                    v = plsc.load_gather(
                        idx_v, [iota * H + (j * 16 * H + h0 + hl)])
                    idxT[pl.ds(blk * HC * HB + hl * HB + j * 16, 16)] = v
                return 0

            lax.fori_loop(0, HC, reorder, 0)

        def start_gather(g, p):
            pltpu.async_copy(
                table_hbm.at[idxT.at[pl.ds(g * rows, rows)]],
                gath.at[pl.ds(p * rows, rows)],
                gsem,
            )

        def start_out(g, p, hh):
            blk = g // steps_per_blk
            hl = (g % steps_per_blk) * GH + hh
            pltpu.async_copy(
                gath.at[pl.ds(p * rows + hh * HB, HB)],
                out_hbm.at[hl, pl.ds((wid * blk_per_w + blk) * HB, HB)],
                osem,
            )

        def wait_bytes(sem, n):
            pltpu.make_async_copy(
                gath.at[pl.ds(0, n)], out_hbm.at[0, pl.ds(0, n)], sem
            ).wait()

        start_gather(0, 0)

        def body(g, _):
            p = lax.rem(g, 3)
            pl.when(g >= 2)(lambda: wait_bytes(osem, rows))
            pl.when(g + 1 < nsteps)(
                lambda: start_gather(g + 1, lax.rem(g + 1, 3)))
            wait_bytes(gsem, rows)
            for hh in range(GH):
                start_out(g, p, hh)
            return 0

        lax.fori_loop(0, nsteps, body, 0)
        wait_bytes(osem, rows)
        wait_bytes(osem, rows)

    return k1


@functools.lru_cache(maxsize=None)
def _make_k2(B: int, H: int, h0: int, aliased: bool):
    NB = 64

    def body(*refs):
        x_ref = refs[0]
        o_ref = refs[-1]
        for k in range(NB):
            x = x_ref[0, pl.ds(k * HB, HB), : DIM]
            o_ref[0, :, k] = _mxu_t(x).reshape(DIM // 8, 8, HB)

    out_shape = jax.ShapeDtypeStruct((H, DIM // 8, B // HB, 8, HB), jnp.float32)
    in_specs = [
        pl.BlockSpec((1, HB * NB, 128), lambda i, j: (i, j, 0)),
    ]
    kwargs = {}
    if aliased:
        in_specs.append(pl.BlockSpec(memory_space=pl.ANY))
        kwargs["input_output_aliases"] = {1: 0}
    return pl.pallas_call(
        body,
        grid=(HC, B // (HB * NB)),
        in_specs=in_specs,
        out_specs=pl.BlockSpec(
            (1, DIM // 8, NB, 8, HB), lambda i, j: (h0 + i, 0, j, 0, 0)
        ),
        out_shape=out_shape,
        compiler_params=pltpu.CompilerParams(
            dimension_semantics=("arbitrary", "arbitrary")
        ),
        **kwargs,
    )


def kernel(input_, weight):
    b, h = input_.shape
    v = weight.shape[0]
    idx = input_.reshape(b * h).astype(jnp.int32)
    t128 = _make_k0(v)(weight.T)
    slabs = [_make_k1(b, h, h0)(idx, t128) for h0 in range(0, h, HC)]
    out5 = _make_k2(b, h, 0, False)(slabs[0])
    for i, h0 in enumerate(range(HC, h, HC)):
        out5 = _make_k2(b, h, h0, True)(slabs[i + 1], out5)
    return out5.transpose(2, 4, 0, 1, 3).reshape(b, h, DIM)
